# Initial kernel scaffold; baseline (speedup 1.0000x reference)
import jax
import jax.numpy as jnp
from jax.experimental import pallas as pl


def kernel(hidden_states, weight):
    raise NotImplementedError("write your pallas kernel here")


# speedup vs baseline: 2.4756x; 2.4756x over previous
import functools

import jax
import jax.numpy as jnp
from jax.experimental import pallas as pl
from jax.experimental.pallas import tpu as pltpu

_HIDDEN = 2048
_N_EXPERTS = 64
_TOP_K = 2
_BSZ = 4
_SEQ = 8192
_ALPHA = 0.1

_BT = 512


def _gate_body(hs_ref, w_ref, idx1_ref, idx2_ref, w1_ref, w2_ref, aux_ref,
               ce_ref, ss_ref):
    step = pl.program_id(0)
    nsteps = pl.num_programs(0)
    blocks_per_batch = _SEQ // _BT

    @pl.when(step == 0)
    def _init():
        ce_ref[...] = jnp.zeros_like(ce_ref)
        ss_ref[...] = jnp.zeros_like(ss_ref)

    x = hs_ref[...]
    w = w_ref[...]
    logits = jax.lax.dot_general(
        x, w, (((1,), (1,)), ((), ())),
        preferred_element_type=jnp.float32)

    m = jnp.max(logits, axis=1, keepdims=True)
    ex = jnp.exp(logits - m)
    scores = ex / jnp.sum(ex, axis=1, keepdims=True)

    i1 = jnp.argmax(scores, axis=1).astype(jnp.int32)
    m1 = jnp.max(scores, axis=1)
    lane = jax.lax.broadcasted_iota(jnp.int32, scores.shape, 1)
    hot1 = lane == i1[:, None]
    masked = jnp.where(hot1, -jnp.inf, scores)
    i2 = jnp.argmax(masked, axis=1).astype(jnp.int32)
    m2 = jnp.max(masked, axis=1)
    hot2 = lane == i2[:, None]

    denom = m1 + m2 + 1e-20
    idx1_ref[...] = i1
    idx2_ref[...] = i2
    w1_ref[...] = m1 / denom
    w2_ref[...] = m2 / denom

    b = step // blocks_per_batch
    cnt = jnp.sum(hot1.astype(jnp.float32) + hot2.astype(jnp.float32),
                  axis=0)
    ce_ref[pl.ds(b, 1), :] += cnt[None, :]
    ss_ref[pl.ds(b, 1), :] += jnp.sum(scores, axis=0)[None, :]

    @pl.when(step == nsteps - 1)
    def _fin():
        ce = ce_ref[...] * (_N_EXPERTS / (_SEQ * _TOP_K))
        sm = ss_ref[...] / _SEQ
        aux_ref[0, 0] = jnp.sum(ce * sm) / _BSZ * _ALPHA


@jax.jit
def kernel(hidden_states, weight):
    bsz, seq, h = hidden_states.shape
    tokens = bsz * seq
    hs = hidden_states.reshape(tokens, h)
    grid = (tokens // _BT,)

    out_shapes = (
        jax.ShapeDtypeStruct((tokens,), jnp.int32),
        jax.ShapeDtypeStruct((tokens,), jnp.int32),
        jax.ShapeDtypeStruct((tokens,), jnp.float32),
        jax.ShapeDtypeStruct((tokens,), jnp.float32),
        jax.ShapeDtypeStruct((1, 1), jnp.float32),
    )
    tok_spec = pl.BlockSpec((_BT,), lambda i: (i,))
    out_specs = (
        tok_spec, tok_spec, tok_spec, tok_spec,
        pl.BlockSpec(memory_space=pltpu.MemorySpace.SMEM),
    )
    in_specs = (
        pl.BlockSpec((_BT, h), lambda i: (i, 0)),
        pl.BlockSpec((_N_EXPERTS, h), lambda i: (0, 0)),
    )
    idx1, idx2, w1, w2, aux = pl.pallas_call(
        _gate_body,
        grid=grid,
        in_specs=in_specs,
        out_specs=out_specs,
        out_shape=out_shapes,
        scratch_shapes=[
            pltpu.VMEM((_BSZ, _N_EXPERTS), jnp.float32),
            pltpu.VMEM((_BSZ, _N_EXPERTS), jnp.float32),
        ],
    )(hs, weight)

    topk_idx = jnp.stack([idx1, idx2], axis=-1)
    topk_weight = jnp.stack([w1, w2], axis=-1)
    return (topk_idx, topk_weight, aux.reshape(()))


# speedup vs baseline: 3.2423x; 1.3097x over previous
import functools

import jax
import jax.numpy as jnp
from jax.experimental import pallas as pl
from jax.experimental.pallas import tpu as pltpu

_HIDDEN = 2048
_N_EXPERTS = 64
_TOP_K = 2
_BSZ = 4
_SEQ = 8192
_ALPHA = 0.1

_BT = 512


def _gate_body(hs_ref, w_ref, idx1_ref, idx2_ref, w1_ref, w2_ref, aux_ref,
               ce_ref, ss_ref):
    step = pl.program_id(0)
    nsteps = pl.num_programs(0)
    blocks_per_batch = _SEQ // _BT

    @pl.when(step == 0)
    def _init():
        ce_ref[...] = jnp.zeros_like(ce_ref)
        ss_ref[...] = jnp.zeros_like(ss_ref)

    x = hs_ref[...]
    w = w_ref[...]
    logits = jax.lax.dot_general(
        w, x, (((1,), (1,)), ((), ())),
        preferred_element_type=jnp.float32)

    m = jnp.max(logits, axis=0, keepdims=True)
    ex = jnp.exp(logits - m)
    scores = ex / jnp.sum(ex, axis=0, keepdims=True)

    i1 = jnp.argmax(scores, axis=0).astype(jnp.int32)
    m1 = jnp.max(scores, axis=0)
    sub = jax.lax.broadcasted_iota(jnp.int32, scores.shape, 0)
    hot1 = sub == i1[None, :]
    masked = jnp.where(hot1, -jnp.inf, scores)
    i2 = jnp.argmax(masked, axis=0).astype(jnp.int32)
    m2 = jnp.max(masked, axis=0)
    hot2 = sub == i2[None, :]

    denom = m1 + m2 + 1e-20
    idx1_ref[...] = i1
    idx2_ref[...] = i2
    w1_ref[...] = m1 / denom
    w2_ref[...] = m2 / denom

    b = step // blocks_per_batch
    bhot = (jax.lax.broadcasted_iota(jnp.int32, (1, _BSZ), 1) == b
            ).astype(jnp.float32)
    cnt = jnp.sum(hot1.astype(jnp.float32) + hot2.astype(jnp.float32),
                  axis=1, keepdims=True)
    ce_ref[...] += cnt * bhot
    ss_ref[...] += jnp.sum(scores, axis=1, keepdims=True) * bhot

    @pl.when(step == nsteps - 1)
    def _fin():
        ce = ce_ref[...] * (_N_EXPERTS / (_SEQ * _TOP_K))
        sm = ss_ref[...] / _SEQ
        aux_ref[0, 0] = jnp.sum(ce * sm) / _BSZ * _ALPHA


@jax.jit
def kernel(hidden_states, weight):
    bsz, seq, h = hidden_states.shape
    tokens = bsz * seq
    hs = hidden_states.reshape(tokens, h)
    grid = (tokens // _BT,)

    out_shapes = (
        jax.ShapeDtypeStruct((tokens,), jnp.int32),
        jax.ShapeDtypeStruct((tokens,), jnp.int32),
        jax.ShapeDtypeStruct((tokens,), jnp.float32),
        jax.ShapeDtypeStruct((tokens,), jnp.float32),
        jax.ShapeDtypeStruct((1, 1), jnp.float32),
    )
    tok_spec = pl.BlockSpec((_BT,), lambda i: (i,))
    out_specs = (
        tok_spec, tok_spec, tok_spec, tok_spec,
        pl.BlockSpec(memory_space=pltpu.MemorySpace.SMEM),
    )
    in_specs = (
        pl.BlockSpec((_BT, h), lambda i: (i, 0)),
        pl.BlockSpec((_N_EXPERTS, h), lambda i: (0, 0)),
    )
    idx1, idx2, w1, w2, aux = pl.pallas_call(
        _gate_body,
        grid=grid,
        in_specs=in_specs,
        out_specs=out_specs,
        out_shape=out_shapes,
        scratch_shapes=[
            pltpu.VMEM((_N_EXPERTS, _BSZ), jnp.float32),
            pltpu.VMEM((_N_EXPERTS, _BSZ), jnp.float32),
        ],
    )(hs, weight)

    topk_idx = jnp.stack([idx1, idx2], axis=-1)
    topk_weight = jnp.stack([w1, w2], axis=-1)
    return (topk_idx, topk_weight, aux.reshape(()))


# speedup vs baseline: 3.9878x; 1.2299x over previous
import functools

import jax
import jax.numpy as jnp
from jax.experimental import pallas as pl
from jax.experimental.pallas import tpu as pltpu

_HIDDEN = 2048
_N_EXPERTS = 64
_TOP_K = 2
_BSZ = 4
_SEQ = 8192
_ALPHA = 0.1

_BT = 1024


def _gate_body(hs_ref, w_ref, idx1_ref, idx2_ref, w1_ref, w2_ref, aux_ref,
               ce_ref, ss_ref):
    step = pl.program_id(0)
    nsteps = pl.num_programs(0)
    blocks_per_batch = _SEQ // _BT

    @pl.when(step == 0)
    def _init():
        ce_ref[...] = jnp.zeros_like(ce_ref)
        ss_ref[...] = jnp.zeros_like(ss_ref)

    x = hs_ref[...]
    w = w_ref[...]
    logits = jax.lax.dot_general(
        w, x, (((1,), (1,)), ((), ())),
        preferred_element_type=jnp.float32)

    m = jnp.max(logits, axis=0, keepdims=True)
    ex = jnp.exp(logits - m)
    scores = ex / jnp.sum(ex, axis=0, keepdims=True)

    i1 = jnp.argmax(scores, axis=0).astype(jnp.int32)
    m1 = jnp.max(scores, axis=0)
    sub = jax.lax.broadcasted_iota(jnp.int32, scores.shape, 0)
    hot1 = sub == i1[None, :]
    masked = jnp.where(hot1, -jnp.inf, scores)
    i2 = jnp.argmax(masked, axis=0).astype(jnp.int32)
    m2 = jnp.max(masked, axis=0)
    hot2 = sub == i2[None, :]

    denom = m1 + m2 + 1e-20
    idx1_ref[...] = i1
    idx2_ref[...] = i2
    w1_ref[...] = m1 / denom
    w2_ref[...] = m2 / denom

    b = step // blocks_per_batch
    bhot = (jax.lax.broadcasted_iota(jnp.int32, (1, _BSZ), 1) == b
            ).astype(jnp.float32)
    cnt = jnp.sum(hot1.astype(jnp.float32) + hot2.astype(jnp.float32),
                  axis=1, keepdims=True)
    ce_ref[...] += cnt * bhot
    ss_ref[...] += jnp.sum(scores, axis=1, keepdims=True) * bhot

    @pl.when(step == nsteps - 1)
    def _fin():
        ce = ce_ref[...] * (_N_EXPERTS / (_SEQ * _TOP_K))
        sm = ss_ref[...] / _SEQ
        aux_ref[0, 0] = jnp.sum(ce * sm) / _BSZ * _ALPHA


@jax.jit
def kernel(hidden_states, weight):
    bsz, seq, h = hidden_states.shape
    tokens = bsz * seq
    hs = hidden_states.reshape(tokens, h)
    grid = (tokens // _BT,)

    out_shapes = (
        jax.ShapeDtypeStruct((tokens,), jnp.int32),
        jax.ShapeDtypeStruct((tokens,), jnp.int32),
        jax.ShapeDtypeStruct((tokens,), jnp.float32),
        jax.ShapeDtypeStruct((tokens,), jnp.float32),
        jax.ShapeDtypeStruct((1, 1), jnp.float32),
    )
    tok_spec = pl.BlockSpec((_BT,), lambda i: (i,))
    out_specs = (
        tok_spec, tok_spec, tok_spec, tok_spec,
        pl.BlockSpec(memory_space=pltpu.MemorySpace.SMEM),
    )
    in_specs = (
        pl.BlockSpec((_BT, h), lambda i: (i, 0)),
        pl.BlockSpec((_N_EXPERTS, h), lambda i: (0, 0)),
    )
    idx1, idx2, w1, w2, aux = pl.pallas_call(
        _gate_body,
        grid=grid,
        in_specs=in_specs,
        out_specs=out_specs,
        out_shape=out_shapes,
        scratch_shapes=[
            pltpu.VMEM((_N_EXPERTS, _BSZ), jnp.float32),
            pltpu.VMEM((_N_EXPERTS, _BSZ), jnp.float32),
        ],
    )(hs, weight)

    topk_idx = jnp.stack([idx1, idx2], axis=-1)
    topk_weight = jnp.stack([w1, w2], axis=-1)
    return (topk_idx, topk_weight, aux.reshape(()))
